# Initial kernel scaffold; baseline (speedup 1.0000x reference)
import jax
import jax.numpy as jnp
from jax.experimental import pallas as pl


def kernel(student_id, exercise_id, theta_w, a_w, b_w):
    raise NotImplementedError("write your pallas kernel here")


# speedup vs baseline: 1.0543x; 1.0543x over previous
import jax
import jax.numpy as jnp
from jax import lax
from jax.experimental import pallas as pl
from jax.experimental.pallas import tpu as pltpu
from jax.experimental.pallas import tpu_sc as plsc

_BATCH = 16384
_K = 128
_NC = 2
_NS = 16
_NW = _NC * _NS
_BPW = _BATCH // _NW
_C = 256
_L = 16


def _mirt_body(sid_hbm, eid_hbm, theta_hbm, a_hbm, b_hbm, out_hbm,
               sid_c, eid_c, eid_v, b_v, theta_v, a_v, part_v, out_v, sem):
    wid = lax.axis_index("s") * _NC + lax.axis_index("c")
    base = wid * _BPW

    pltpu.sync_copy(eid_hbm.at[pl.ds(base, _BPW)], eid_v)
    b_cp = pltpu.async_copy(b_hbm.at[eid_v], b_v, sem)

    for c in range(_BPW // _C):
        pltpu.sync_copy(sid_hbm.at[pl.ds(base + c * _C, _C)], sid_c)
        pltpu.sync_copy(eid_hbm.at[pl.ds(base + c * _C, _C)], eid_c)
        t_cp = pltpu.async_copy(theta_hbm.at[sid_c], theta_v, sem)
        a_cp = pltpu.async_copy(a_hbm.at[eid_c], a_v, sem)
        t_cp.wait()
        a_cp.wait()

        def item(i, carry, c=c):
            acc = jnp.zeros((_L,), jnp.float32)
            for k in range(_K // _L):
                av = a_v[i, pl.ds(k * _L, _L)]
                tv = theta_v[i, pl.ds(k * _L, _L)]
                acc = acc + tv / (1.0 + jnp.exp(-av))
            part_v[pl.ds((c * _C + i) * _L, _L)] = acc
            return carry

        lax.fori_loop(0, _C, item, 0)

    b_cp.wait()
    lanes = lax.iota(jnp.int32, _L)

    def group(g, carry):
        row0 = g * _L
        flat0 = row0 * _L + lanes * _L
        acc = jnp.zeros((_L,), jnp.float32)
        for j in range(_L):
            acc = acc + plsc.load_gather(part_v, [flat0 + j])
        bv = b_v[pl.ds(row0, _L)]
        out_v[pl.ds(row0, _L)] = 1.0 / (1.0 + jnp.exp(bv - acc))
        return carry

    lax.fori_loop(0, _BPW // _L, group, 0)

    pltpu.sync_copy(out_v, out_hbm.at[pl.ds(base, _BPW)])


def _mirt_call(student_id, exercise_id, theta_w, a_w, b_flat, interpret=False):
    mesh = plsc.VectorSubcoreMesh(core_axis_name="c", subcore_axis_name="s",
                                  num_cores=_NC, num_subcores=_NS)
    run = pl.kernel(
        _mirt_body,
        out_type=jax.ShapeDtypeStruct((_BATCH,), jnp.float32),
        mesh=mesh,
        scratch_types=[
            pltpu.VMEM((_C,), jnp.int32),
            pltpu.VMEM((_C,), jnp.int32),
            pltpu.VMEM((_BPW,), jnp.int32),
            pltpu.VMEM((_BPW,), jnp.float32),
            pltpu.VMEM((_C, _K), jnp.float32),
            pltpu.VMEM((_C, _K), jnp.float32),
            pltpu.VMEM((_BPW * _L,), jnp.float32),
            pltpu.VMEM((_BPW,), jnp.float32),
            pltpu.SemaphoreType.DMA,
        ],
        compiler_params=pltpu.CompilerParams(needs_layout_passes=False),
        interpret=interpret,
    )
    return run(student_id, exercise_id, theta_w, a_w, b_flat)


def kernel(student_id, exercise_id, theta_w, a_w, b_w):
    return _mirt_call(student_id, exercise_id, theta_w, a_w,
                      b_w.reshape((-1,)))


# speedup vs baseline: 1.3687x; 1.2982x over previous
import jax
import jax.numpy as jnp
from jax import lax
from jax.experimental import pallas as pl
from jax.experimental.pallas import tpu as pltpu
from jax.experimental.pallas import tpu_sc as plsc

_BATCH = 16384
_K = 128
_NC = 2
_NS = 16
_NW = _NC * _NS
_BPW = _BATCH // _NW
_C = 128
_NCHUNK = _BPW // _C
_L = 16


def _mirt_body(sid_hbm, eid_hbm, theta_hbm, a_hbm, b_hbm, out_hbm,
               sid_c0, eid_c0, theta_v0, a_v0,
               sid_c1, eid_c1, theta_v1, a_v1,
               eid_v, b_v, part_v, out_v,
               sem0, sem1, semb):
    wid = lax.axis_index("s") * _NC + lax.axis_index("c")
    base = wid * _BPW

    pltpu.sync_copy(eid_hbm.at[pl.ds(base, _BPW)], eid_v)
    b_cp = pltpu.async_copy(b_hbm.at[eid_v], b_v, semb)

    slots = ((sid_c0, eid_c0, theta_v0, a_v0, sem0),
             (sid_c1, eid_c1, theta_v1, a_v1, sem1))

    def issue(c):
        sid_b, eid_b, th_b, a_b, sem = slots[c % 2]
        pltpu.sync_copy(sid_hbm.at[pl.ds(base + c * _C, _C)], sid_b)
        pltpu.sync_copy(eid_hbm.at[pl.ds(base + c * _C, _C)], eid_b)
        t_cp = pltpu.async_copy(theta_hbm.at[sid_b], th_b, sem)
        a_cp = pltpu.async_copy(a_hbm.at[eid_b], a_b, sem)
        return t_cp, a_cp

    pending = [None, None]
    pending[0] = issue(0)

    for c in range(_NCHUNK):
        if c + 1 < _NCHUNK:
            pending[(c + 1) % 2] = issue(c + 1)
        t_cp, a_cp = pending[c % 2]
        t_cp.wait()
        a_cp.wait()
        _, _, th_b, a_b, _ = slots[c % 2]

        @plsc.parallel_loop(0, _C, unroll=4)
        def _item(i, th_b=th_b, a_b=a_b, c=c):
            acc = jnp.zeros((_L,), jnp.float32)
            for k in range(_K // _L):
                av = a_b[i, pl.ds(k * _L, _L)]
                tv = th_b[i, pl.ds(k * _L, _L)]
                acc = acc + tv / (1.0 + jnp.exp(-av))
            part_v[pl.ds((c * _C + i) * _L, _L)] = acc

    b_cp.wait()
    lanes = lax.iota(jnp.int32, _L)

    @plsc.parallel_loop(0, _BPW // _L, unroll=2)
    def _group(g):
        row0 = g * _L
        flat0 = row0 * _L + lanes * _L
        acc = jnp.zeros((_L,), jnp.float32)
        for j in range(_L):
            acc = acc + plsc.load_gather(part_v, [flat0 + j])
        bv = b_v[pl.ds(row0, _L)]
        out_v[pl.ds(row0, _L)] = 1.0 / (1.0 + jnp.exp(bv - acc))

    pltpu.sync_copy(out_v, out_hbm.at[pl.ds(base, _BPW)])


def _mirt_call(student_id, exercise_id, theta_w, a_w, b_flat, interpret=False):
    mesh = plsc.VectorSubcoreMesh(core_axis_name="c", subcore_axis_name="s",
                                  num_cores=_NC, num_subcores=_NS)
    chunk_slot = [
        pltpu.VMEM((_C,), jnp.int32),
        pltpu.VMEM((_C,), jnp.int32),
        pltpu.VMEM((_C, _K), jnp.float32),
        pltpu.VMEM((_C, _K), jnp.float32),
    ]
    run = pl.kernel(
        _mirt_body,
        out_type=jax.ShapeDtypeStruct((_BATCH,), jnp.float32),
        mesh=mesh,
        scratch_types=chunk_slot + chunk_slot + [
            pltpu.VMEM((_BPW,), jnp.int32),
            pltpu.VMEM((_BPW,), jnp.float32),
            pltpu.VMEM((_BPW * _L,), jnp.float32),
            pltpu.VMEM((_BPW,), jnp.float32),
            pltpu.SemaphoreType.DMA,
            pltpu.SemaphoreType.DMA,
            pltpu.SemaphoreType.DMA,
        ],
        compiler_params=pltpu.CompilerParams(needs_layout_passes=False),
        interpret=interpret,
    )
    return run(student_id, exercise_id, theta_w, a_w, b_flat)


def kernel(student_id, exercise_id, theta_w, a_w, b_w):
    return _mirt_call(student_id, exercise_id, theta_w, a_w,
                      b_w.reshape((-1,)))


# speedup vs baseline: 1.4054x; 1.0268x over previous
import jax
import jax.numpy as jnp
from jax import lax
from jax.experimental import pallas as pl
from jax.experimental.pallas import tpu as pltpu
from jax.experimental.pallas import tpu_sc as plsc

_BATCH = 16384
_K = 128
_NC = 2
_NS = 16
_NW = _NC * _NS
_BPW = _BATCH // _NW
_C = 128
_NCHUNK = _BPW // _C
_L = 16


def _mirt_body(sid_hbm, eid_hbm, theta_hbm, a_hbm, b_hbm, out_hbm,
               theta_v0, a_v0, theta_v1, a_v1,
               sid_v, eid_v, b_v, part_v, out_v,
               sem0, sem1, semb):
    wid = lax.axis_index("s") * _NC + lax.axis_index("c")
    base = wid * _BPW

    pltpu.sync_copy(sid_hbm.at[pl.ds(base, _BPW)], sid_v)
    pltpu.sync_copy(eid_hbm.at[pl.ds(base, _BPW)], eid_v)

    slots = ((theta_v0, a_v0, sem0), (theta_v1, a_v1, sem1))

    def issue(c):
        th_b, a_b, sem = slots[c % 2]
        t_cp = pltpu.async_copy(theta_hbm.at[sid_v.at[pl.ds(c * _C, _C)]],
                                th_b, sem)
        a_cp = pltpu.async_copy(a_hbm.at[eid_v.at[pl.ds(c * _C, _C)]],
                                a_b, sem)
        return t_cp, a_cp

    pending = [None, None]
    pending[0] = issue(0)
    b_cp = pltpu.async_copy(b_hbm.at[eid_v], b_v, semb)

    for c in range(_NCHUNK):
        if c + 1 < _NCHUNK:
            pending[(c + 1) % 2] = issue(c + 1)
        t_cp, a_cp = pending[c % 2]
        t_cp.wait()
        a_cp.wait()
        th_b, a_b, _ = slots[c % 2]

        @plsc.parallel_loop(0, _C, unroll=4)
        def _item(i, th_b=th_b, a_b=a_b, c=c):
            acc = jnp.zeros((_L,), jnp.float32)
            for k in range(_K // _L):
                av = a_b[i, pl.ds(k * _L, _L)]
                tv = th_b[i, pl.ds(k * _L, _L)]
                acc = acc + tv / (1.0 + jnp.exp(-av))
            part_v[pl.ds((c * _C + i) * _L, _L)] = acc

    b_cp.wait()
    lanes = lax.iota(jnp.int32, _L)

    @plsc.parallel_loop(0, _BPW // _L, unroll=2)
    def _group(g):
        row0 = g * _L
        flat0 = row0 * _L + lanes * _L
        acc = jnp.zeros((_L,), jnp.float32)
        for j in range(_L):
            acc = acc + plsc.load_gather(part_v, [flat0 + j])
        bv = b_v[pl.ds(row0, _L)]
        out_v[pl.ds(row0, _L)] = 1.0 / (1.0 + jnp.exp(bv - acc))

    pltpu.sync_copy(out_v, out_hbm.at[pl.ds(base, _BPW)])


def _mirt_call(student_id, exercise_id, theta_w, a_w, b_flat, interpret=False):
    mesh = plsc.VectorSubcoreMesh(core_axis_name="c", subcore_axis_name="s",
                                  num_cores=_NC, num_subcores=_NS)
    chunk_slot = [
        pltpu.VMEM((_C, _K), jnp.float32),
        pltpu.VMEM((_C, _K), jnp.float32),
    ]
    run = pl.kernel(
        _mirt_body,
        out_type=jax.ShapeDtypeStruct((_BATCH,), jnp.float32),
        mesh=mesh,
        scratch_types=chunk_slot + chunk_slot + [
            pltpu.VMEM((_BPW,), jnp.int32),
            pltpu.VMEM((_BPW,), jnp.int32),
            pltpu.VMEM((_BPW,), jnp.float32),
            pltpu.VMEM((_BPW * _L,), jnp.float32),
            pltpu.VMEM((_BPW,), jnp.float32),
            pltpu.SemaphoreType.DMA,
            pltpu.SemaphoreType.DMA,
            pltpu.SemaphoreType.DMA,
        ],
        compiler_params=pltpu.CompilerParams(needs_layout_passes=False),
        interpret=interpret,
    )
    return run(student_id, exercise_id, theta_w, a_w, b_flat)


def kernel(student_id, exercise_id, theta_w, a_w, b_w):
    return _mirt_call(student_id, exercise_id, theta_w, a_w,
                      b_w.reshape((-1,)))
